# Initial kernel scaffold; baseline (speedup 1.0000x reference)
import jax
import jax.numpy as jnp
from jax.experimental import pallas as pl


def kernel(x, c, W1, b1, W2, b2, W3, b3, train):
    raise NotImplementedError("write your pallas kernel here")


# speedup vs baseline: 6.5752x; 6.5752x over previous
import functools

import jax
import jax.numpy as jnp
from jax.experimental import pallas as pl
from jax.experimental.pallas import tpu as pltpu

_KNOTS = 16
_BOUND = 5.0
_LOWER = 3
_TN = 4096


def _cumsum16(v):
    tn = v.shape[1]
    for s in (1, 2, 4, 8):
        z = jnp.zeros((s, tn), v.dtype)
        v = v + jnp.concatenate([z, v[: 16 - s, :]], axis=0)
    return v


def _softmax10(logits):
    m = jnp.max(logits, axis=0, keepdims=True)
    e = jnp.exp(logits - m)
    s = jnp.sum(e, axis=0, keepdims=True)
    return e * ((2.0 * _BOUND) / s)


def _spline_body(h_ref, lo_ref, w1_ref, b1_ref, w2_ref, b2_ref, w3_ref,
                 b3_ref, y_ref, ld_ref):
    h = h_ref[...]
    a = jnp.dot(w1_ref[...], h, preferred_element_type=jnp.float32)
    a = jnp.maximum(a + b1_ref[...], 0.0)
    a = jnp.dot(w2_ref[...], a, preferred_element_type=jnp.float32)
    a = jnp.maximum(a + b2_ref[...], 0.0)
    p = jnp.dot(w3_ref[...], a, preferred_element_type=jnp.float32)
    p = p + b3_ref[...]

    lo = lo_ref[...]
    tn = lo.shape[1]
    kpos = jax.lax.broadcasted_iota(jnp.int32, (_KNOTS, tn), 0).astype(
        jnp.float32)

    ys = []
    lds = []
    for d in range(_LOWER):
        lw = p[16 * d:16 * d + 16, :]
        lh = p[48 + 16 * d:48 + 16 * d + 16, :]
        lg = p[96 + 16 * d:96 + 16 * d + 16, :]
        Wd = _softmax10(lw)
        Hd = _softmax10(lh)
        Dd = jnp.maximum(lg, 0.0) + jnp.log1p(jnp.exp(-jnp.abs(lg)))
        cw = _cumsum16(Wd)

        xd = lo[d:d + 1, :]
        oob = (xd <= -_BOUND) | (xd >= _BOUND)
        xm = jnp.where(oob, -_BOUND, xd)

        mk = jnp.where(xm >= cw - _BOUND, 1.0, 0.0)
        idxf = jnp.clip(jnp.sum(mk, axis=0, keepdims=True), 0.0, 15.0)
        seq = kpos == idxf
        slt = kpos < idxf
        sm1 = kpos == idxf - 1.0

        def msum(mask, v):
            return jnp.sum(jnp.where(mask, v, 0.0), axis=0, keepdims=True)

        wk = msum(seq, Wd)
        hk = msum(seq, Hd)
        xkb = msum(slt, Wd) - _BOUND
        ykb = msum(slt, Hd) - _BOUND
        dkb = jnp.where(idxf == 0.0, 1.0, msum(sm1, Dd))
        dk1 = jnp.where(idxf == 15.0, 1.0, msum(seq, Dd))

        rw = 1.0 / wk
        sk = hk * rw
        relx = jnp.clip((xm - xkb) * rw, 0.0, 1.0)
        omr = 1.0 - relx
        r1 = relx * omr
        den = sk + (dk1 + dkb - 2.0 * sk) * r1
        num = hk * (sk * relx * relx + dkb * r1)
        y = ykb + num / den
        ld = (2.0 * jnp.log(sk)
              + jnp.log(dk1 * relx * relx + 2.0 * sk * r1 + dkb * omr * omr)
              - 2.0 * jnp.log(den))
        ys.append(jnp.where(oob, xd, y))
        lds.append(jnp.where(oob, 0.0, ld))

    y_ref[0:3, :] = jnp.concatenate(ys, axis=0)
    y_ref[3:6, :] = h[0:3, :]
    ld_ref[...] = lds[0] + lds[1] + lds[2]


@functools.partial(jax.jit, static_argnames=("interpret",))
def kernel(x, c, W1, b1, W2, b2, W3, b3, train, interpret=False):
    n = x.shape[0]
    g = -(-n // _TN)
    npad = g * _TN - n

    upT = jnp.transpose(x[:, _LOWER:])
    loT = jnp.transpose(x[:, :_LOWER])
    hT = jnp.concatenate([upT, jnp.transpose(c),
                          jnp.zeros((1, n), jnp.float32)], axis=0)
    hT = jnp.pad(hT, ((0, 0), (0, npad)))
    loT = jnp.pad(loT, ((0, 0), (0, npad)))

    w1t = jnp.pad(jnp.transpose(W1), ((0, 0), (0, 1)))
    w2t = jnp.transpose(W2)
    w3t = jnp.transpose(W3)
    z1 = jnp.zeros((1, 128), jnp.float32)
    rows = []
    brows = []
    for off, width in ((0, 16), (16, 16), (32, 15)):
        for d in range(_LOWER):
            s = 47 * d + off
            rows.append(w3t[s:s + width])
            brows.append(b3[s:s + width])
            if width == 15:
                rows.append(z1)
                brows.append(jnp.zeros((1,), jnp.float32))
    w3g = jnp.concatenate(rows, axis=0)
    b3g = jnp.concatenate(brows, axis=0).reshape(144, 1)

    np_tot = n + npad
    yT, ldT = pl.pallas_call(
        _spline_body,
        grid=(g,),
        in_specs=[
            pl.BlockSpec((8, _TN), lambda i: (0, i)),
            pl.BlockSpec((3, _TN), lambda i: (0, i)),
            pl.BlockSpec((128, 8), lambda i: (0, 0)),
            pl.BlockSpec((128, 1), lambda i: (0, 0)),
            pl.BlockSpec((128, 128), lambda i: (0, 0)),
            pl.BlockSpec((128, 1), lambda i: (0, 0)),
            pl.BlockSpec((144, 128), lambda i: (0, 0)),
            pl.BlockSpec((144, 1), lambda i: (0, 0)),
        ],
        out_specs=[
            pl.BlockSpec((6, _TN), lambda i: (0, i)),
            pl.BlockSpec((1, _TN), lambda i: (0, i)),
        ],
        out_shape=[
            jax.ShapeDtypeStruct((6, np_tot), jnp.float32),
            jax.ShapeDtypeStruct((1, np_tot), jnp.float32),
        ],
        compiler_params=pltpu.CompilerParams(
            dimension_semantics=("parallel",),
        ),
        interpret=interpret,
    )(hT, loT, w1t, b1.reshape(128, 1), w2t, b2.reshape(128, 1), w3g, b3g)

    y = jnp.transpose(yT[:, :n])
    log_det = ldT[0, :n]
    return y, log_det
